# baseline (speedup 1.0000x reference)
import functools

import jax
import jax.numpy as jnp
from jax import lax
from jax.experimental import pallas as pl
from jax.experimental.pallas import tpu as pltpu

BATCH = 4096
DIM = 4096
BLK = 32
GRID = BATCH // BLK


def _tc_body(uidx_ref, iidx_ref, u_tab, i_tab, u_out, i_out, usem, isem):
    g = pl.program_id(0)

    ucopies = []
    icopies = []
    for r in range(BLK):
        urow = uidx_ref[g * BLK + r]
        cp = pltpu.make_async_copy(u_tab.at[pl.ds(urow, 1)],
                                   u_out.at[pl.ds(r, 1)], usem)
        cp.start()
        ucopies.append(cp)
        irow = iidx_ref[g * BLK + r]
        cp = pltpu.make_async_copy(i_tab.at[pl.ds(irow, 1)],
                                   i_out.at[pl.ds(r, 1)], isem)
        cp.start()
        icopies.append(cp)
    for cp in ucopies:
        cp.wait()
    for cp in icopies:
        cp.wait()


@jax.jit
def _tc_gather(users, items, user_table, item_table):
    grid_spec = pltpu.PrefetchScalarGridSpec(
        num_scalar_prefetch=2,
        grid=(GRID,),
        in_specs=[
            pl.BlockSpec(memory_space=pl.ANY),
            pl.BlockSpec(memory_space=pl.ANY),
        ],
        out_specs=[
            pl.BlockSpec((BLK, DIM), lambda g, u, i: (g, 0)),
            pl.BlockSpec((BLK, DIM), lambda g, u, i: (g, 0)),
        ],
        scratch_shapes=[pltpu.SemaphoreType.DMA, pltpu.SemaphoreType.DMA],
    )
    return pl.pallas_call(
        _tc_body,
        grid_spec=grid_spec,
        out_shape=[
            jax.ShapeDtypeStruct((BATCH, DIM), jnp.float32),
            jax.ShapeDtypeStruct((BATCH, DIM), jnp.float32),
        ],
    )(users, items, user_table, item_table)


def kernel(users, items, user_table, item_table):
    u_repr, i_repr = _tc_gather(users, items, user_table, item_table)
    return (u_repr, i_repr)
